# Initial kernel scaffold; baseline (speedup 1.0000x reference)
import jax
import jax.numpy as jnp
from jax.experimental import pallas as pl


def kernel(atom_batch, xyz, p1, W, b):
    raise NotImplementedError("write your pallas kernel here")


# speedup vs baseline: 1.4148x; 1.4148x over previous
import functools

import jax
import jax.numpy as jnp
from jax import lax
from jax.experimental import pallas as pl
from jax.experimental.pallas import tpu as pltpu
from jax.experimental.pallas import tpu_sc as plsc

_N = 320000
_D = 128
_S = 10000
_SP = 10240
_ACC = 3 * _SP
_NC = 2
_NS = 16
_NW = _NC * _NS
_APW = _N // _NW
_EPW = 3 * _APW
_CH = 6000
_BT = 2000


def _tc_body(p1_ref, xyz_ref, ab_ref, w_ref, b_ref, upd_ref, idx_ref):
    q = jnp.dot(p1_ref[...], w_ref[...], preferred_element_type=jnp.float32)
    upd_ref[...] = (q + b_ref[0]) * xyz_ref[...]
    lane = jax.lax.broadcasted_iota(jnp.int32, (_BT, 3), 1)
    idx_ref[...] = ab_ref[...] * 3 + lane


def _tc_stage(p1, xyz, ab1, W, b):
    return pl.pallas_call(
        _tc_body,
        grid=(_N // _BT,),
        in_specs=[
            pl.BlockSpec((_BT, _D), lambda i: (i, 0)),
            pl.BlockSpec((_BT, 3), lambda i: (i, 0)),
            pl.BlockSpec((_BT, 1), lambda i: (i, 0)),
            pl.BlockSpec((_D, 1), lambda i: (0, 0)),
            pl.BlockSpec(memory_space=pltpu.SMEM),
        ],
        out_specs=[
            pl.BlockSpec((_BT, 3), lambda i: (i, 0)),
            pl.BlockSpec((_BT, 3), lambda i: (i, 0)),
        ],
        out_shape=[
            jax.ShapeDtypeStruct((_N, 3), jnp.float32),
            jax.ShapeDtypeStruct((_N, 3), jnp.int32),
        ],
    )(p1, xyz, ab1, W, b)


_sc_mesh = plsc.VectorSubcoreMesh(core_axis_name="c", subcore_axis_name="s")


@functools.partial(
    pl.kernel,
    out_type=jax.ShapeDtypeStruct((_NC * _ACC,), jnp.float32),
    mesh=_sc_mesh,
    scratch_types=[
        pltpu.VMEM((_EPW,), jnp.float32),
        pltpu.VMEM((_CH,), jnp.int32),
        pltpu.VMEM_SHARED((_ACC,), jnp.float32),
    ],
    compiler_params=pltpu.CompilerParams(use_tc_tiling_on_sc=False),
)
def _sc_scatter(upd_hbm, idx_hbm, zeros_hbm, out_hbm, upd_v, idx_blk, acc):
    cid = lax.axis_index("c")
    sid = lax.axis_index("s")
    wid = cid * _NS + sid
    ept = _ACC // _NS

    pltpu.sync_copy(zeros_hbm.at[pl.ds(sid * ept, ept)],
                    acc.at[pl.ds(sid * ept, ept)])
    plsc.subcore_barrier()

    base = wid * _EPW
    pltpu.sync_copy(upd_hbm.at[pl.ds(base, _EPW)], upd_v)
    for chunk in range(_EPW // _CH):
        pltpu.sync_copy(idx_hbm.at[pl.ds(base + chunk * _CH, _CH)], idx_blk)
        pltpu.sync_copy(upd_v.at[pl.ds(chunk * _CH, _CH)],
                        acc.at[idx_blk], add=True)

    plsc.subcore_barrier()
    pltpu.sync_copy(acc.at[pl.ds(sid * ept, ept)],
                    out_hbm.at[pl.ds(cid * _ACC + sid * ept, ept)])


def kernel(atom_batch, xyz, p1, W, b):
    ab1 = atom_batch.astype(jnp.int32).reshape(_N, 1)
    upd, idx3 = _tc_stage(p1, xyz, ab1, W, b)
    zeros = jnp.zeros((_ACC,), jnp.float32)
    part = _sc_scatter(upd.reshape(-1), idx3.reshape(-1), zeros)
    acc = part[:_ACC] + part[_ACC:]
    return acc.reshape(_SP, 3)[:_S]


# speedup vs baseline: 1.9568x; 1.3831x over previous
import functools

import jax
import jax.numpy as jnp
from jax import lax
from jax.experimental import pallas as pl
from jax.experimental.pallas import tpu as pltpu
from jax.experimental.pallas import tpu_sc as plsc

_N = 320000
_D = 128
_S = 10000
_SP = 10240
_NC = 2
_NS = 16
_NW = _NC * _NS
_APW = _N // _NW
_BT = 512


def _tc_body(p1_ref, xyz_ref, w_ref, b_ref, ux_ref, uy_ref, uz_ref):
    q = jnp.dot(p1_ref[...], w_ref[...], preferred_element_type=jnp.float32)
    upd = (q + b_ref[0]) * xyz_ref[...]
    updt = upd.T
    ux_ref[...] = updt[0]
    uy_ref[...] = updt[1]
    uz_ref[...] = updt[2]


def _tc_stage(p1, xyz, W, b):
    return pl.pallas_call(
        _tc_body,
        grid=(_N // _BT,),
        in_specs=[
            pl.BlockSpec((_BT, _D), lambda i: (i, 0)),
            pl.BlockSpec((_BT, 3), lambda i: (i, 0)),
            pl.BlockSpec((_D, 1), lambda i: (0, 0)),
            pl.BlockSpec(memory_space=pltpu.SMEM),
        ],
        out_specs=[
            pl.BlockSpec((_BT,), lambda i: (i,)),
            pl.BlockSpec((_BT,), lambda i: (i,)),
            pl.BlockSpec((_BT,), lambda i: (i,)),
        ],
        out_shape=[
            jax.ShapeDtypeStruct((_N,), jnp.float32),
            jax.ShapeDtypeStruct((_N,), jnp.float32),
            jax.ShapeDtypeStruct((_N,), jnp.float32),
        ],
    )(p1, xyz, W, b)


_sc_mesh = plsc.VectorSubcoreMesh(core_axis_name="c", subcore_axis_name="s")


@functools.partial(
    pl.kernel,
    out_type=jax.ShapeDtypeStruct((_NC * 3 * _SP,), jnp.float32),
    mesh=_sc_mesh,
    scratch_types=[
        pltpu.VMEM((_APW,), jnp.float32),
        pltpu.VMEM((_APW,), jnp.float32),
        pltpu.VMEM((_APW,), jnp.float32),
        pltpu.VMEM((_APW,), jnp.int32),
        pltpu.VMEM_SHARED((_SP,), jnp.float32),
        pltpu.VMEM_SHARED((_SP,), jnp.float32),
        pltpu.VMEM_SHARED((_SP,), jnp.float32),
    ],
    compiler_params=pltpu.CompilerParams(use_tc_tiling_on_sc=False),
)
def _sc_scatter(ux_hbm, uy_hbm, uz_hbm, ab_hbm, zeros_hbm, out_hbm,
                ux_v, uy_v, uz_v, idx_v, accx, accy, accz):
    cid = lax.axis_index("c")
    sid = lax.axis_index("s")
    wid = cid * _NS + sid
    ept = _SP // _NS

    for acc in (accx, accy, accz):
        pltpu.sync_copy(zeros_hbm.at[pl.ds(sid * ept, ept)],
                        acc.at[pl.ds(sid * ept, ept)])
    plsc.subcore_barrier()

    base = wid * _APW
    pltpu.sync_copy(ab_hbm.at[pl.ds(base, _APW)], idx_v)
    pltpu.sync_copy(ux_hbm.at[pl.ds(base, _APW)], ux_v)
    pltpu.sync_copy(uy_hbm.at[pl.ds(base, _APW)], uy_v)
    pltpu.sync_copy(uz_hbm.at[pl.ds(base, _APW)], uz_v)
    pltpu.sync_copy(ux_v, accx.at[idx_v], add=True)
    pltpu.sync_copy(uy_v, accy.at[idx_v], add=True)
    pltpu.sync_copy(uz_v, accz.at[idx_v], add=True)

    plsc.subcore_barrier()
    for k, acc in enumerate((accx, accy, accz)):
        pltpu.sync_copy(acc.at[pl.ds(sid * ept, ept)],
                        out_hbm.at[pl.ds((cid * 3 + k) * _SP + sid * ept, ept)])


def kernel(atom_batch, xyz, p1, W, b):
    ux, uy, uz = _tc_stage(p1, xyz, W, b)
    zeros = jnp.zeros((_SP,), jnp.float32)
    part = _sc_scatter(ux, uy, uz, atom_batch.astype(jnp.int32), zeros)
    o = part.reshape(_NC, 3, _SP)
    comps = o[0] + o[1]
    return comps[:, :_S].T


# speedup vs baseline: 10.2548x; 5.2407x over previous
import functools

import jax
import jax.numpy as jnp
from jax import lax
from jax.experimental import pallas as pl
from jax.experimental.pallas import tpu as pltpu
from jax.experimental.pallas import tpu_sc as plsc

_N = 320000
_D = 128
_S = 10000
_SP = 10240
_NC = 2
_NS = 16
_NW = _NC * _NS
_APW = _N // _NW
_BT = 20480
_NP = 327680


def _tc_body(p1_ref, xyzt_ref, w_ref, b_ref, ux_ref, uy_ref, uz_ref):
    q = jnp.dot(p1_ref[...], w_ref[...], preferred_element_type=jnp.float32)
    q_row = q.T[0] + b_ref[0]
    ux_ref[...] = q_row * xyzt_ref[0]
    uy_ref[...] = q_row * xyzt_ref[1]
    uz_ref[...] = q_row * xyzt_ref[2]


def _tc_stage(p1, xyzt, W, b):
    return pl.pallas_call(
        _tc_body,
        grid=(_NP // _BT,),
        in_specs=[
            pl.BlockSpec((_BT, _D), lambda i: (i, 0)),
            pl.BlockSpec((3, _BT), lambda i: (0, i)),
            pl.BlockSpec((_D, 1), lambda i: (0, 0)),
            pl.BlockSpec(memory_space=pltpu.SMEM),
        ],
        out_specs=[
            pl.BlockSpec((_BT,), lambda i: (i,)),
            pl.BlockSpec((_BT,), lambda i: (i,)),
            pl.BlockSpec((_BT,), lambda i: (i,)),
        ],
        out_shape=[
            jax.ShapeDtypeStruct((_NP,), jnp.float32),
            jax.ShapeDtypeStruct((_NP,), jnp.float32),
            jax.ShapeDtypeStruct((_NP,), jnp.float32),
        ],
    )(p1, xyzt, W, b)


_sc_mesh = plsc.VectorSubcoreMesh(core_axis_name="c", subcore_axis_name="s")


@functools.partial(
    pl.kernel,
    out_type=jax.ShapeDtypeStruct((_NC * 3 * _SP,), jnp.float32),
    mesh=_sc_mesh,
    scratch_types=[
        pltpu.VMEM((_APW,), jnp.float32),
        pltpu.VMEM((_APW,), jnp.float32),
        pltpu.VMEM((_APW,), jnp.float32),
        pltpu.VMEM((_APW,), jnp.int32),
        pltpu.VMEM_SHARED((_SP,), jnp.float32),
        pltpu.VMEM_SHARED((_SP,), jnp.float32),
        pltpu.VMEM_SHARED((_SP,), jnp.float32),
    ],
    compiler_params=pltpu.CompilerParams(use_tc_tiling_on_sc=False),
)
def _sc_scatter(ux_hbm, uy_hbm, uz_hbm, ab_hbm, zeros_hbm, out_hbm,
                ux_v, uy_v, uz_v, idx_v, accx, accy, accz):
    cid = lax.axis_index("c")
    sid = lax.axis_index("s")
    wid = cid * _NS + sid
    ept = _SP // _NS

    for acc in (accx, accy, accz):
        pltpu.sync_copy(zeros_hbm.at[pl.ds(sid * ept, ept)],
                        acc.at[pl.ds(sid * ept, ept)])
    plsc.subcore_barrier()

    base = wid * _APW
    pltpu.sync_copy(ab_hbm.at[pl.ds(base, _APW)], idx_v)
    pltpu.sync_copy(ux_hbm.at[pl.ds(base, _APW)], ux_v)
    pltpu.sync_copy(uy_hbm.at[pl.ds(base, _APW)], uy_v)
    pltpu.sync_copy(uz_hbm.at[pl.ds(base, _APW)], uz_v)
    pltpu.sync_copy(ux_v, accx.at[idx_v], add=True)
    pltpu.sync_copy(uy_v, accy.at[idx_v], add=True)
    pltpu.sync_copy(uz_v, accz.at[idx_v], add=True)

    plsc.subcore_barrier()
    for k, acc in enumerate((accx, accy, accz)):
        pltpu.sync_copy(acc.at[pl.ds(sid * ept, ept)],
                        out_hbm.at[pl.ds((cid * 3 + k) * _SP + sid * ept, ept)])


def kernel(atom_batch, xyz, p1, W, b):
    ux, uy, uz = _tc_stage(p1, xyz.T, W, b)
    zeros = jnp.zeros((_SP,), jnp.float32)
    part = _sc_scatter(ux, uy, uz, atom_batch.astype(jnp.int32), zeros)
    o = part.reshape(_NC, 3, _SP)
    comps = o[0] + o[1]
    return comps[:, :_S].T


# speedup vs baseline: 10.8653x; 1.0595x over previous
import functools

import jax
import jax.numpy as jnp
from jax import lax
from jax.experimental import pallas as pl
from jax.experimental.pallas import tpu as pltpu
from jax.experimental.pallas import tpu_sc as plsc

_N = 320000
_D = 128
_S = 10000
_SP = 10240
_NC = 2
_NS = 16
_NW = _NC * _NS
_APW = _N // _NW
_BT = 40960
_NP = 327680


def _tc_body(p1_ref, xyzt_ref, w_ref, b_ref, ux_ref, uy_ref, uz_ref):
    q = jnp.dot(p1_ref[...], w_ref[...], preferred_element_type=jnp.float32)
    q_row = q.T[0] + b_ref[0]
    ux_ref[...] = q_row * xyzt_ref[0]
    uy_ref[...] = q_row * xyzt_ref[1]
    uz_ref[...] = q_row * xyzt_ref[2]


def _tc_stage(p1, xyzt, W, b):
    return pl.pallas_call(
        _tc_body,
        grid=(_NP // _BT,),
        in_specs=[
            pl.BlockSpec((_BT, _D), lambda i: (i, 0)),
            pl.BlockSpec((3, _BT), lambda i: (0, i)),
            pl.BlockSpec((_D, 1), lambda i: (0, 0)),
            pl.BlockSpec(memory_space=pltpu.SMEM),
        ],
        out_specs=[
            pl.BlockSpec((_BT,), lambda i: (i,)),
            pl.BlockSpec((_BT,), lambda i: (i,)),
            pl.BlockSpec((_BT,), lambda i: (i,)),
        ],
        out_shape=[
            jax.ShapeDtypeStruct((_NP,), jnp.float32),
            jax.ShapeDtypeStruct((_NP,), jnp.float32),
            jax.ShapeDtypeStruct((_NP,), jnp.float32),
        ],
    )(p1, xyzt, W, b)


_sc_mesh = plsc.VectorSubcoreMesh(core_axis_name="c", subcore_axis_name="s")


@functools.partial(
    pl.kernel,
    out_type=jax.ShapeDtypeStruct((_NC * 3 * _SP,), jnp.float32),
    mesh=_sc_mesh,
    scratch_types=[
        pltpu.VMEM((_APW,), jnp.float32),
        pltpu.VMEM((_APW,), jnp.float32),
        pltpu.VMEM((_APW,), jnp.float32),
        pltpu.VMEM((_APW,), jnp.int32),
        pltpu.VMEM_SHARED((_SP,), jnp.float32),
        pltpu.VMEM_SHARED((_SP,), jnp.float32),
        pltpu.VMEM_SHARED((_SP,), jnp.float32),
        pltpu.SemaphoreType.DMA,
        pltpu.SemaphoreType.DMA,
    ],
    compiler_params=pltpu.CompilerParams(use_tc_tiling_on_sc=False),
)
def _sc_scatter(ux_hbm, uy_hbm, uz_hbm, ab_hbm, zeros_hbm, out_hbm,
                ux_v, uy_v, uz_v, idx_v, accx, accy, accz, sem1, sem2):
    cid = lax.axis_index("c")
    sid = lax.axis_index("s")
    wid = cid * _NS + sid
    ept = _SP // _NS
    base = wid * _APW

    c1 = pltpu.async_copy(ab_hbm.at[pl.ds(base, _APW)], idx_v, sem1)
    c2 = pltpu.async_copy(ux_hbm.at[pl.ds(base, _APW)], ux_v, sem1)
    c3 = pltpu.async_copy(uy_hbm.at[pl.ds(base, _APW)], uy_v, sem1)
    c4 = pltpu.async_copy(uz_hbm.at[pl.ds(base, _APW)], uz_v, sem1)
    for acc in (accx, accy, accz):
        pltpu.sync_copy(zeros_hbm.at[pl.ds(sid * ept, ept)],
                        acc.at[pl.ds(sid * ept, ept)])
    plsc.subcore_barrier()
    c1.wait()
    c2.wait()
    c3.wait()
    c4.wait()

    s1 = pltpu.async_copy(ux_v, accx.at[idx_v], sem2, add=True)
    s2 = pltpu.async_copy(uy_v, accy.at[idx_v], sem2, add=True)
    s3 = pltpu.async_copy(uz_v, accz.at[idx_v], sem2, add=True)
    s1.wait()
    s2.wait()
    s3.wait()

    plsc.subcore_barrier()
    for k, acc in enumerate((accx, accy, accz)):
        pltpu.sync_copy(acc.at[pl.ds(sid * ept, ept)],
                        out_hbm.at[pl.ds((cid * 3 + k) * _SP + sid * ept, ept)])


def kernel(atom_batch, xyz, p1, W, b):
    ux, uy, uz = _tc_stage(p1, xyz.T, W, b)
    zeros = jnp.zeros((_SP,), jnp.float32)
    part = _sc_scatter(ux, uy, uz, atom_batch.astype(jnp.int32), zeros)
    o = part.reshape(_NC, 3, _SP)
    comps = o[0] + o[1]
    return comps[:, :_S].T
